# baseline (speedup 1.0000x reference)
import jax
import jax.numpy as jnp
from jax.experimental import pallas as pl
from jax.experimental.pallas import tpu as pltpu
from jax.experimental.pallas import tpu_sc as plsc

_WIN = 128
_BLK = 512
_TB = 512


def _sc_gather(idx, W, N, D):
    mesh = plsc.VectorSubcoreMesh(core_axis_name="core",
                                  subcore_axis_name="subcore")

    @pl.kernel(out_type=jax.ShapeDtypeStruct((N, D), W.dtype), mesh=mesh,
               compiler_params=pltpu.CompilerParams(use_tc_tiling_on_sc=False),
               scratch_types=[pltpu.SemaphoreType.DMA])
    def gather_kernel(w_hbm, i_hbm, o_hbm, sem):
        def body(i_vmem, o_vmem):
            copies = [
                pltpu.async_copy(
                    w_hbm.at[i_vmem.at[0, pl.ds(j * _WIN, _WIN)]],
                    o_vmem.at[pl.ds(j * _WIN, _WIN)],
                    sem,
                )
                for j in range(_BLK // _WIN)
            ]
            for c in copies:
                c.wait()

        pltpu.emit_pipeline(
            body,
            grid=(N // _BLK,),
            in_specs=[pl.BlockSpec((1, _BLK), index_map=lambda i: (0, i))],
            out_specs=[pl.BlockSpec((_BLK, D), index_map=lambda i: (i, 0))],
            core_axis_name=("core", "subcore"),
            dimension_semantics=(pltpu.PARALLEL,),
        )(i_hbm, o_hbm)

    return gather_kernel(W, idx)


def _tc_transpose(mid2d, H, B, D):
    bpr = 128 // D
    rows = _TB // bpr

    def body(in_ref, out_ref):
        t3 = in_ref[...].reshape(rows, bpr, D)
        out_ref[0] = jnp.transpose(t3, (2, 0, 1)).reshape(D, _TB)

    grid = (H * B // _TB,)
    blocks_per_h = B // _TB
    return pl.pallas_call(
        body,
        grid=grid,
        in_specs=[pl.BlockSpec((rows, 128), lambda g: (g, 0))],
        out_specs=pl.BlockSpec((1, D, _TB),
                               lambda g: (g // blocks_per_h, 0,
                                          g % blocks_per_h)),
        out_shape=jax.ShapeDtypeStruct((H, D, B), jnp.float32),
    )(mid2d)


def kernel(x, W):
    B, H = x.shape
    V, D = W.shape
    N = B * H

    idx = jnp.transpose(x).reshape(1, N)
    mid = _sc_gather(idx, W, N, D)
    mid2d = mid.reshape(N * D // 128, 128)
    out3 = _tc_transpose(mid2d, H, B, D)
    return jnp.transpose(out3, (2, 0, 1))


# speedup vs baseline: 4.2352x; 4.2352x over previous
import jax
import jax.numpy as jnp
from jax.experimental import pallas as pl
from jax.experimental.pallas import tpu as pltpu
from jax.experimental.pallas import tpu_sc as plsc

_WIN = 128
_BLK = 512
_TS = 512


def _sc_gather(idx, W, N, D):
    mesh = plsc.VectorSubcoreMesh(core_axis_name="core",
                                  subcore_axis_name="subcore")

    @pl.kernel(out_type=jax.ShapeDtypeStruct((N, D), W.dtype), mesh=mesh,
               compiler_params=pltpu.CompilerParams(use_tc_tiling_on_sc=False),
               scratch_types=[pltpu.SemaphoreType.DMA])
    def gather_kernel(w_hbm, i_hbm, o_hbm, sem):
        def body(i_vmem, o_vmem):
            copies = [
                pltpu.async_copy(
                    w_hbm.at[i_vmem.at[0, pl.ds(j * _WIN, _WIN)]],
                    o_vmem.at[pl.ds(j * _WIN, _WIN)],
                    sem,
                )
                for j in range(_BLK // _WIN)
            ]
            for c in copies:
                c.wait()

        pltpu.emit_pipeline(
            body,
            grid=(N // _BLK,),
            in_specs=[pl.BlockSpec((1, _BLK), index_map=lambda i: (0, i))],
            out_specs=[pl.BlockSpec((_BLK, D), index_map=lambda i: (i, 0))],
            core_axis_name=("core", "subcore"),
            dimension_semantics=(pltpu.PARALLEL,),
        )(i_hbm, o_hbm)

    return gather_kernel(W, idx)


def _tc_transpose(mid2d, H, B, D):
    mb = B // 2 // _TS
    cb = B // _TS

    def body(in_ref, out_ref):
        p = pl.program_id(0) % cb // mb
        blk = in_ref[...]

        @pl.when(p == 0)
        def _():
            out_ref[0] = blk[:, :D].T

        @pl.when(p == 1)
        def _():
            out_ref[0] = blk[:, D:].T

    return pl.pallas_call(
        body,
        grid=(H * cb,),
        in_specs=[pl.BlockSpec(
            (_TS, 128),
            lambda g: ((g // cb) * mb + g % cb % mb, 0))],
        out_specs=pl.BlockSpec(
            (1, D, _TS), lambda g: (g // cb, 0, g % cb)),
        out_shape=jax.ShapeDtypeStruct((H, D, B), jnp.float32),
    )(mid2d)


def kernel(x, W):
    B, H = x.shape
    V, D = W.shape
    N = B * H

    xt = jnp.transpose(x)
    xperm = jnp.transpose(xt.reshape(H, 2, B // 2), (0, 2, 1))
    idx = xperm.reshape(1, N)

    mid = _sc_gather(idx, W, N, D)
    mid2d = mid.reshape(N * D // 128, 128)
    out3 = _tc_transpose(mid2d, H, B, D)
    return jnp.transpose(out3, (2, 0, 1))


# speedup vs baseline: 7.1365x; 1.6850x over previous
import jax
import jax.numpy as jnp
from jax.experimental import pallas as pl
from jax.experimental.pallas import tpu as pltpu
from jax.experimental.pallas import tpu_sc as plsc

_WIN = 128
_BLK = 512
_TS = 8192


def _sc_gather(idx, W, N, D):
    mesh = plsc.VectorSubcoreMesh(core_axis_name="core",
                                  subcore_axis_name="subcore")

    @pl.kernel(out_type=jax.ShapeDtypeStruct((N, D), W.dtype), mesh=mesh,
               compiler_params=pltpu.CompilerParams(use_tc_tiling_on_sc=False),
               scratch_types=[pltpu.SemaphoreType.DMA])
    def gather_kernel(w_hbm, i_hbm, o_hbm, sem):
        def body(i_vmem, o_vmem):
            copies = [
                pltpu.async_copy(
                    w_hbm.at[i_vmem.at[0, pl.ds(j * _WIN, _WIN)]],
                    o_vmem.at[pl.ds(j * _WIN, _WIN)],
                    sem,
                )
                for j in range(_BLK // _WIN)
            ]
            for c in copies:
                c.wait()

        pltpu.emit_pipeline(
            body,
            grid=(N // _BLK,),
            in_specs=[pl.BlockSpec((1, _BLK), index_map=lambda i: (0, i))],
            out_specs=[pl.BlockSpec((_BLK, D), index_map=lambda i: (i, 0))],
            core_axis_name=("core", "subcore"),
            dimension_semantics=(pltpu.PARALLEL,),
        )(i_hbm, o_hbm)

    return gather_kernel(W, idx)


def _tc_transpose(mid2d, H, B, D):
    mb = B // 2 // _TS
    cb = B // _TS

    def body(in_ref, out_ref):
        p = pl.program_id(0) % cb // mb
        blk = in_ref[...]

        @pl.when(p == 0)
        def _():
            out_ref[0] = blk[:, :D].T

        @pl.when(p == 1)
        def _():
            out_ref[0] = blk[:, D:].T

    return pl.pallas_call(
        body,
        grid=(H * cb,),
        in_specs=[pl.BlockSpec(
            (_TS, 128),
            lambda g: ((g // cb) * mb + g % cb % mb, 0))],
        out_specs=pl.BlockSpec(
            (1, D, _TS), lambda g: (g // cb, 0, g % cb)),
        out_shape=jax.ShapeDtypeStruct((H, D, B), jnp.float32),
    )(mid2d)


def kernel(x, W):
    B, H = x.shape
    V, D = W.shape
    N = B * H

    xt = jnp.transpose(x)
    xperm = jnp.transpose(xt.reshape(H, 2, B // 2), (0, 2, 1))
    idx = xperm.reshape(1, N)

    mid = _sc_gather(idx, W, N, D)
    mid2d = mid.reshape(N * D // 128, 128)
    out3 = _tc_transpose(mid2d, H, B, D)
    return jnp.transpose(out3, (2, 0, 1))
